# baseline (speedup 1.0000x reference)
import jax
import jax.numpy as jnp
from jax.experimental import pallas as pl


_BLOCK_ROWS = 4000


def _gconv_mm_kernel(x_ref, w_ref, b_ref, o_ref):
    o_ref[...] = (
        jnp.dot(x_ref[...], w_ref[...], preferred_element_type=jnp.float32)
        + b_ref[...]
    )


def kernel(inputs, state, weight, biases):
    del state
    batch = inputs.shape[0]
    in_size, out_dim = weight.shape
    rows = inputs.size // in_size
    x = inputs.reshape(rows, in_size)
    b2 = biases.reshape(1, out_dim)

    grid = rows // _BLOCK_ROWS
    out = pl.pallas_call(
        _gconv_mm_kernel,
        grid=(grid,),
        in_specs=[
            pl.BlockSpec((_BLOCK_ROWS, in_size), lambda i: (i, 0)),
            pl.BlockSpec((in_size, out_dim), lambda i: (0, 0)),
            pl.BlockSpec((1, out_dim), lambda i: (0, 0)),
        ],
        out_specs=pl.BlockSpec((_BLOCK_ROWS, out_dim), lambda i: (i, 0)),
        out_shape=jax.ShapeDtypeStruct((rows, out_dim), jnp.float32),
    )(x, weight, b2)

    return out.reshape(batch, -1)


# speedup vs baseline: 3.1473x; 3.1473x over previous
import jax
import jax.numpy as jnp
from jax.experimental import pallas as pl


_NODES_PER_BLOCK = 250


def _gconv_kernel(x_ref, w_ref, b_ref, o_ref):
    w = w_ref[...]
    b = b_ref[...]
    k2 = w.shape[0]
    n2 = w.shape[1]

    def body(c, _):
        x = x_ref[:, pl.ds(c * k2, k2)]
        o_ref[:, pl.ds(c * n2, n2)] = (
            jnp.dot(x, w, preferred_element_type=jnp.float32) + b
        )
        return 0

    jax.lax.fori_loop(0, _NODES_PER_BLOCK // 2, body, 0)


def kernel(inputs, state, weight, biases):
    del state
    batch = inputs.shape[0]
    in_size, out_dim = weight.shape
    nodes = inputs.shape[1] // in_size

    zeros = jnp.zeros_like(weight)
    w2 = jnp.block([[weight, zeros], [zeros, weight]])
    b2 = jnp.tile(biases.reshape(1, out_dim), (1, 2))

    grid = nodes // _NODES_PER_BLOCK
    return pl.pallas_call(
        _gconv_kernel,
        grid=(grid,),
        in_specs=[
            pl.BlockSpec((batch, _NODES_PER_BLOCK * in_size), lambda i: (0, i)),
            pl.BlockSpec((2 * in_size, 2 * out_dim), lambda i: (0, 0)),
            pl.BlockSpec((1, 2 * out_dim), lambda i: (0, 0)),
        ],
        out_specs=pl.BlockSpec((batch, _NODES_PER_BLOCK * out_dim), lambda i: (0, i)),
        out_shape=jax.ShapeDtypeStruct((batch, nodes * out_dim), jnp.float32),
    )(inputs, w2, b2)
